# Initial kernel scaffold; baseline (speedup 1.0000x reference)
import jax
import jax.numpy as jnp
from jax.experimental import pallas as pl


def kernel(x, L):
    raise NotImplementedError("write your pallas kernel here")


# speedup vs baseline: 10.0707x; 10.0707x over previous
import jax
import jax.numpy as jnp
from jax.experimental import pallas as pl
from jax.experimental.pallas import tpu as pltpu

_K = 16
_EPSILON = 0.5
_LAM = 0.1
_BETA = 0.1
_NSEL = _K + 2


def _pick_rows(b):
    for r in (256, 200, 128, 80, 64, 40, 32, 16, 8):
        if b % r == 0:
            return r
    return b


def _stage1_body(h_ref, hb_ref, l_ref, vals_ref, idx_ref, attn0_ref):
    h = h_ref[...]
    hb = hb_ref[...]
    lb = l_ref[...]
    g = jnp.dot(hb, h.T, preferred_element_type=jnp.float32)
    xx_b = jnp.sum(hb * hb, axis=1, keepdims=True)
    yy = jnp.sum(h * h, axis=1)[None, :]
    mn_g = jnp.min(g, axis=1, keepdims=True)
    mx_g = jnp.max(g, axis=1, keepdims=True)
    gn = (g - mn_g) / (mx_g - mn_g + 1e-8)
    dist = jnp.sqrt(jnp.clip(xx_b + yy - 2.0 * g, 1e-12, None))
    d1 = dist - 2.0 * _BETA * lb - 1e-5 * gn
    mn1 = jnp.min(d1, axis=1, keepdims=True)
    mx1 = jnp.max(d1, axis=1, keepdims=True)
    d = jnp.maximum((d1 - mn1) / (mx1 - mn1 + 1e-8), 0.0)
    cols = jax.lax.broadcasted_iota(jnp.int32, d.shape, 1)
    for m in range(_NSEL):
        v = jnp.min(d, axis=1)
        j = jnp.argmin(d, axis=1).astype(jnp.int32)
        vals_ref[:, m:m + 1] = v[:, None]
        mask = cols == j[:, None]
        if 1 <= m <= _K:
            idx_ref[:, m - 1:m] = j[:, None]
            attn0_ref[:, m - 1:m] = jnp.sum(
                jnp.where(mask, gn, 0.0), axis=1, keepdims=True)
        d = jnp.where(mask, 2.0, d)


def _stage1(h, l):
    b, c = h.shape
    r = _pick_rows(b)
    grid = (b // r,)
    return pl.pallas_call(
        _stage1_body,
        grid=grid,
        in_specs=[
            pl.BlockSpec((b, c), lambda i: (0, 0)),
            pl.BlockSpec((r, c), lambda i: (i, 0)),
            pl.BlockSpec((r, b), lambda i: (i, 0)),
        ],
        out_specs=[
            pl.BlockSpec((r, _NSEL), lambda i: (i, 0)),
            pl.BlockSpec((r, _K), lambda i: (i, 0)),
            pl.BlockSpec((r, _K), lambda i: (i, 0)),
        ],
        out_shape=[
            jax.ShapeDtypeStruct((b, _NSEL), jnp.float32),
            jax.ShapeDtypeStruct((b, _K), jnp.int32),
            jax.ShapeDtypeStruct((b, _K), jnp.float32),
        ],
    )(h, h, l)


def _apply_body(idx_ref, a_ref, h_ref, hb_ref, out_ref):
    idx = idx_ref[...]
    a = a_ref[...]
    h = h_ref[...]
    hb = hb_ref[...]
    r = idx.shape[0]
    b = h.shape[0]
    cols = jax.lax.broadcasted_iota(jnp.int32, (r, b), 1)
    p = jnp.zeros((r, b), jnp.float32)
    for k in range(_K):
        p = p + jnp.where(cols == idx[:, k:k + 1], a[:, k:k + 1], 0.0)
    out = jnp.dot(p, h, preferred_element_type=jnp.float32)
    out_ref[...] = _EPSILON * out + (1.0 - _EPSILON) * hb


def _apply(idx, a, h):
    b, c = h.shape
    r = _pick_rows(b)
    grid = (b // r,)
    return pl.pallas_call(
        _apply_body,
        grid=grid,
        in_specs=[
            pl.BlockSpec((r, _K), lambda i: (i, 0)),
            pl.BlockSpec((r, _K), lambda i: (i, 0)),
            pl.BlockSpec((b, c), lambda i: (0, 0)),
            pl.BlockSpec((r, c), lambda i: (i, 0)),
        ],
        out_specs=pl.BlockSpec((r, c), lambda i: (i, 0)),
        out_shape=jax.ShapeDtypeStruct((b, c), jnp.float32),
    )(idx, a, h, h)


def _stats_body(f_ref, fb_ref, idx_ref, attn_ref):
    f = f_ref[...]
    fb = fb_ref[...]
    idx = idx_ref[...]
    g = jnp.dot(fb, f.T, preferred_element_type=jnp.float32)
    mn = jnp.min(g, axis=1, keepdims=True)
    mx = jnp.max(g, axis=1, keepdims=True)
    gn = (g - mn) / (mx - mn + 1e-8)
    cols = jax.lax.broadcasted_iota(jnp.int32, g.shape, 1)
    for k in range(_K):
        mask = cols == idx[:, k:k + 1]
        attn_ref[:, k:k + 1] = jnp.sum(
            jnp.where(mask, gn, 0.0), axis=1, keepdims=True)


def _stats(f, idx):
    b, c = f.shape
    r = _pick_rows(b)
    grid = (b // r,)
    return pl.pallas_call(
        _stats_body,
        grid=grid,
        in_specs=[
            pl.BlockSpec((b, c), lambda i: (0, 0)),
            pl.BlockSpec((r, c), lambda i: (i, 0)),
            pl.BlockSpec((r, _K), lambda i: (i, 0)),
        ],
        out_specs=pl.BlockSpec((r, _K), lambda i: (i, 0)),
        out_shape=jax.ShapeDtypeStruct((b, _K), jnp.float32),
    )(f, f, idx)


def kernel(x, L):
    h = x[0]
    vals, idx, attn0 = _stage1(h, L)
    dval = vals[:, 1:_K + 1]
    dk = vals[:, _K + 1]
    gamma = jnp.mean(0.5 * (_K * dk - jnp.sum(dval, axis=1)))
    inv2g = 1.0 / (2.0 * gamma + 1e-8)
    eta = (1.0 / _K) * (1.0 + jnp.sum(dval, axis=1) * inv2g)

    a0 = jax.nn.relu(eta[:, None] - (dval - _LAM * attn0) * inv2g)
    fm1 = _apply(idx, a0, h)

    attn1 = _stats(fm1, idx)
    a1 = jax.nn.relu(eta[:, None] - (dval - _LAM * attn1) * inv2g)
    fm2 = _apply(idx, a1, h)
    return fm2[None, :, :]


# speedup vs baseline: 14.3247x; 1.4224x over previous
import functools

import jax
import jax.numpy as jnp
from jax import lax
from jax.experimental import pallas as pl
from jax.experimental.pallas import tpu as pltpu
from jax.experimental.pallas import tpu_sc as plsc

_K = 16
_EPSILON = 0.5
_LAM = 0.1
_BETA = 0.1
_NSEL = _K + 2

_NW = 32
_LANES = 16


def _pick_rows(b):
    for r in (256, 200, 128, 80, 64, 40, 32, 16, 8):
        if b % r == 0:
            return r
    return b


def _stage1_body(h_ref, hb_ref, l_ref, vals_ref, idx_ref, mnmx_ref):
    h = h_ref[...]
    hb = hb_ref[...]
    lb = l_ref[...]
    g = jnp.dot(hb, h.T, preferred_element_type=jnp.float32)
    xx_b = jnp.sum(hb * hb, axis=1, keepdims=True)
    yy = jnp.sum(h * h, axis=1)[None, :]
    mn_g = jnp.min(g, axis=1, keepdims=True)
    mx_g = jnp.max(g, axis=1, keepdims=True)
    mnmx_ref[:, 0:1] = mn_g
    mnmx_ref[:, 1:2] = mx_g
    gn = (g - mn_g) / (mx_g - mn_g + 1e-8)
    dist = jnp.sqrt(jnp.clip(xx_b + yy - 2.0 * g, 1e-12, None))
    d1 = dist - 2.0 * _BETA * lb - 1e-5 * gn
    mn1 = jnp.min(d1, axis=1, keepdims=True)
    mx1 = jnp.max(d1, axis=1, keepdims=True)
    d = jnp.maximum((d1 - mn1) / (mx1 - mn1 + 1e-8), 0.0)
    cols = jax.lax.broadcasted_iota(jnp.int32, d.shape, 1)
    for m in range(_NSEL):
        v = jnp.min(d, axis=1)
        j = jnp.argmin(d, axis=1).astype(jnp.int32)
        vals_ref[:, m:m + 1] = v[:, None]
        if 1 <= m <= _K:
            idx_ref[:, m - 1:m] = j[:, None]
        if m < _NSEL - 1:
            d = jnp.where(cols == j[:, None], 2.0, d)


def _stage1(h, l):
    b, c = h.shape
    r = _pick_rows(b)
    grid = (b // r,)
    return pl.pallas_call(
        _stage1_body,
        grid=grid,
        in_specs=[
            pl.BlockSpec((b, c), lambda i: (0, 0)),
            pl.BlockSpec((r, c), lambda i: (i, 0)),
            pl.BlockSpec((r, b), lambda i: (i, 0)),
        ],
        out_specs=[
            pl.BlockSpec((r, _NSEL), lambda i: (i, 0)),
            pl.BlockSpec((r, _K), lambda i: (i, 0)),
            pl.BlockSpec((r, 2), lambda i: (i, 0)),
        ],
        out_shape=[
            jax.ShapeDtypeStruct((b, _NSEL), jnp.float32),
            jax.ShapeDtypeStruct((b, _K), jnp.int32),
            jax.ShapeDtypeStruct((b, 2), jnp.float32),
        ],
    )(h, h, l)


def _stats_body(f_ref, fb_ref, mnmx_ref):
    f = f_ref[...]
    fb = fb_ref[...]
    g = jnp.dot(fb, f.T, preferred_element_type=jnp.float32)
    mnmx_ref[:, 0:1] = jnp.min(g, axis=1, keepdims=True)
    mnmx_ref[:, 1:2] = jnp.max(g, axis=1, keepdims=True)


def _stats(f):
    b, c = f.shape
    r = _pick_rows(b)
    return pl.pallas_call(
        _stats_body,
        grid=(b // r,),
        in_specs=[
            pl.BlockSpec((b, c), lambda i: (0, 0)),
            pl.BlockSpec((r, c), lambda i: (i, 0)),
        ],
        out_specs=pl.BlockSpec((r, 2), lambda i: (i, 0)),
        out_shape=jax.ShapeDtypeStruct((b, 2), jnp.float32),
    )(f, f)


_KW = 24


def _sc_apply_body(shared_tables, c, rw, refs):
    if shared_tables:
        (sum_hbm, idx_hbm, c0_hbm, c1_hbm, c2_hbm, out_hbm,
         idxv, c0v, c1v, c2v, obuf, gsum, sems) = refs
        dot_hbm, gdot = sum_hbm, gsum
    else:
        (sum_hbm, dot_hbm, idx_hbm, c0_hbm, c1_hbm, c2_hbm, out_hbm,
         idxv, c0v, c1v, c2v, obuf, gsum, gdot, sems) = refs
    nchunk = c // _LANES
    wid = lax.axis_index("s") * 2 + lax.axis_index("c")
    base = wid * rw
    pltpu.sync_copy(idx_hbm.at[pl.ds(base, rw)], idxv)
    pltpu.sync_copy(c0_hbm.at[pl.ds(base, rw)], c0v)
    pltpu.sync_copy(c1_hbm.at[pl.ds(base, rw)], c1v)
    pltpu.sync_copy(c2_hbm.at[pl.ds(base, rw)], c2v)

    iot = lax.iota(jnp.int32, _LANES)

    def fire(r, slot):
        pltpu.async_copy(sum_hbm.at[idxv.at[r]], gsum.at[slot], sems[slot])
        if not shared_tables:
            pltpu.async_copy(dot_hbm.at[idxv.at[r]], gdot.at[slot],
                             sems[2 + slot])

    def drain(r, slot):
        pltpu.make_async_copy(sum_hbm.at[idxv.at[r]], gsum.at[slot],
                              sems[slot]).wait()
        if not shared_tables:
            pltpu.make_async_copy(dot_hbm.at[idxv.at[r]], gdot.at[slot],
                                  sems[2 + slot]).wait()

    def compute_row(r, slot):
        own_d = [gdot[slot, 0, pl.ds(ch * _LANES, _LANES)]
                 for ch in range(nchunk)]
        dots = jnp.zeros((_LANES,), jnp.float32)
        for k in range(_K):
            acc = own_d[0] * gdot[slot, 1 + k, pl.ds(0, _LANES)]
            for ch in range(1, nchunk):
                acc = acc + own_d[ch] * gdot[
                    slot, 1 + k, pl.ds(ch * _LANES, _LANES)]
            dots = jnp.where(iot == k, jnp.sum(acc), dots)
        c0row = c0v[r, :]
        c1row = c1v[r, :]
        c2row = c2v[r, :]
        a = jnp.maximum(c0row + c1row * (dots - c2row), 0.0)
        oacc = [jnp.zeros((_LANES,), jnp.float32) for _ in range(nchunk)]
        for k in range(_K):
            ab = jnp.sum(jnp.where(iot == k, a, 0.0))
            for ch in range(nchunk):
                vec = gsum[slot, 1 + k, pl.ds(ch * _LANES, _LANES)]
                oacc[ch] = oacc[ch] + ab * vec
        @pl.when(r >= 2)
        def _():
            pltpu.make_async_copy(obuf.at[slot], out_hbm.at[base + r - 2],
                                  sems[-2 + slot]).wait()
        for ch in range(nchunk):
            own = gsum[slot, 0, pl.ds(ch * _LANES, _LANES)]
            res = _EPSILON * oacc[ch] + (1.0 - _EPSILON) * own
            obuf[slot, pl.ds(ch * _LANES, _LANES)] = res
        pltpu.async_copy(obuf.at[slot], out_hbm.at[base + r],
                         sems[-2 + slot])

    fire(0, 0)

    def step(i, carry):
        r0 = 2 * i
        fire(r0 + 1, 1)
        drain(r0, 0)
        compute_row(r0, 0)
        fire(jnp.minimum(r0 + 2, rw - 1), 0)
        drain(r0 + 1, 1)
        compute_row(r0 + 1, 1)
        return carry

    lax.fori_loop(0, rw // 2, step, 0)
    drain(rw - 1, 0)
    for slot in range(2):
        pltpu.make_async_copy(obuf.at[slot],
                              out_hbm.at[base + rw - 2 + slot],
                              sems[-2 + slot]).wait()


def _sc_apply(sum_tab, dot_tab, idx, c0, c1, c2, shared_tables):
    bp, c = sum_tab.shape
    rw = bp // _NW
    mesh = plsc.VectorSubcoreMesh(core_axis_name="c", subcore_axis_name="s")
    scratch = [
        pltpu.VMEM((rw, _KW), jnp.int32),
        pltpu.VMEM((rw, _K), jnp.float32),
        pltpu.VMEM((rw, _K), jnp.float32),
        pltpu.VMEM((rw, _K), jnp.float32),
        pltpu.VMEM((2, c), jnp.float32),
        pltpu.VMEM((2, _KW, c), jnp.float32),
    ]
    if not shared_tables:
        scratch.append(pltpu.VMEM((2, _KW, c), jnp.float32))
    nsem = 4 if shared_tables else 6
    for _ in range(nsem):
        scratch.append(pltpu.SemaphoreType.DMA)

    def body(*refs):
        if shared_tables:
            (sum_hbm, idx_hbm, c0_hbm, c1_hbm, c2_hbm, out_hbm,
             idxv, c0v, c1v, c2v, obuf, gsum, s0, s1, o0, o1) = refs
            _sc_apply_body(True, c, rw,
                           (sum_hbm, idx_hbm, c0_hbm, c1_hbm, c2_hbm,
                            out_hbm, idxv, c0v, c1v, c2v, obuf, gsum,
                            [s0, s1, o0, o1]))
        else:
            (sum_hbm, dot_hbm, idx_hbm, c0_hbm, c1_hbm, c2_hbm, out_hbm,
             idxv, c0v, c1v, c2v, obuf, gsum, gdot,
             s0, s1, s2, s3, o0, o1) = refs
            _sc_apply_body(False, c, rw,
                           (sum_hbm, dot_hbm, idx_hbm, c0_hbm, c1_hbm,
                            c2_hbm, out_hbm, idxv, c0v, c1v, c2v, obuf,
                            gsum, gdot, [s0, s1, s2, s3, o0, o1]))

    kern = functools.partial(
        pl.kernel, mesh=mesh,
        out_type=jax.ShapeDtypeStruct((bp, c), jnp.float32),
        scratch_types=scratch,
        compiler_params=pltpu.CompilerParams(
            needs_layout_passes=False, use_tc_tiling_on_sc=False),
    )(body)
    if shared_tables:
        return kern(sum_tab, idx, c0, c1, c2)
    return kern(sum_tab, dot_tab, idx, c0, c1, c2)


def _pad_rows(a, bp):
    pad = [(0, bp - a.shape[0])] + [(0, 0)] * (a.ndim - 1)
    return jnp.pad(a, pad)


def kernel(x, L):
    h = x[0]
    b, c = h.shape
    bp = ((b + 8 * _NW - 1) // (8 * _NW)) * (8 * _NW)
    vals, idx, mnmx0 = _stage1(h, L)
    dval = vals[:, 1:_K + 1]
    dk = vals[:, _K + 1]
    gamma = jnp.mean(0.5 * (_K * dk - jnp.sum(dval, axis=1)))
    inv2g = 1.0 / (2.0 * gamma + 1e-8)
    eta = (1.0 / _K) * (1.0 + jnp.sum(dval, axis=1) * inv2g)

    c0 = eta[:, None] - inv2g * dval
    hp = _pad_rows(h, bp)
    own = jnp.arange(b, dtype=jnp.int32)[:, None]
    idxw = jnp.concatenate(
        [own, idx, jnp.broadcast_to(own, (b, _KW - _K - 1))], axis=1)
    idxp = _pad_rows(idxw, bp)
    c0p = _pad_rows(c0, bp)

    def consts(mnmx):
        c1 = inv2g * _LAM / (mnmx[:, 1] - mnmx[:, 0] + 1e-8)
        c1k = jnp.broadcast_to(c1[:, None], (b, _K))
        c2k = jnp.broadcast_to(mnmx[:, 0:1], (b, _K))
        return _pad_rows(c1k, bp), _pad_rows(c2k, bp)

    c1p, c2p = consts(mnmx0)
    fm1p = _sc_apply(hp, hp, idxp, c0p, c1p, c2p, shared_tables=True)

    mnmx1 = _stats(fm1p[:b])
    c1p, c2p = consts(mnmx1)
    fm2p = _sc_apply(hp, fm1p, idxp, c0p, c1p, c2p, shared_tables=False)
    return fm2p[:b][None, :, :]


# speedup vs baseline: 18.9867x; 1.3255x over previous
import functools

import jax
import jax.numpy as jnp
from jax import lax
from jax.experimental import pallas as pl
from jax.experimental.pallas import tpu as pltpu
from jax.experimental.pallas import tpu_sc as plsc

_K = 16
_EPSILON = 0.5
_LAM = 0.1
_BETA = 0.1
_NSEL = _K + 2

_NW = 32
_LANES = 16


def _pick_rows(b):
    for r in (256, 200, 128, 80, 64, 40, 32, 16, 8):
        if b % r == 0:
            return r
    return b


def _stage1_body(h_ref, hb_ref, l_ref, vals_ref, idx_ref, mnmx_ref):
    h = h_ref[...]
    hb = hb_ref[...]
    lb = l_ref[...]
    g = jnp.dot(hb, h.T, preferred_element_type=jnp.float32)
    xx_b = jnp.sum(hb * hb, axis=1, keepdims=True)
    yy = jnp.sum(h * h, axis=1)[None, :]
    mn_g = jnp.min(g, axis=1, keepdims=True)
    mx_g = jnp.max(g, axis=1, keepdims=True)
    mnmx_ref[:, 0:1] = mn_g
    mnmx_ref[:, 1:2] = mx_g
    gn = (g - mn_g) / (mx_g - mn_g + 1e-8)
    dist = jnp.sqrt(jnp.clip(xx_b + yy - 2.0 * g, 1e-12, None))
    d1 = dist - 2.0 * _BETA * lb - 1e-5 * gn
    mn1 = jnp.min(d1, axis=1, keepdims=True)
    mx1 = jnp.max(d1, axis=1, keepdims=True)
    d = jnp.maximum((d1 - mn1) / (mx1 - mn1 + 1e-8), 0.0)
    q = d.shape[1] // 4
    m1, m2, m3, m4 = (
        jax.lax.bitcast_convert_type(
            (jax.lax.bitcast_convert_type(
                d[:, s * q:(s + 1) * q], jnp.int32) & ~3) | s,
            jnp.float32)
        for s in range(4))
    m1, m2 = jnp.minimum(m1, m2), jnp.maximum(m1, m2)
    m3, m4 = jnp.minimum(m3, m4), jnp.maximum(m3, m4)
    m1, m3 = jnp.minimum(m1, m3), jnp.maximum(m1, m3)
    m2, m4 = jnp.minimum(m2, m4), jnp.maximum(m2, m4)
    m2, m3 = jnp.minimum(m2, m3), jnp.maximum(m2, m3)
    cols = jax.lax.broadcasted_iota(jnp.int32, m1.shape, 1)
    for m in range(_NSEL):
        vmf = jnp.min(m1, axis=1)
        j = jnp.argmin(m1, axis=1).astype(jnp.int32)
        vmi = jax.lax.bitcast_convert_type(vmf, jnp.int32)
        v = jax.lax.bitcast_convert_type(vmi & ~3, jnp.float32)
        vals_ref[:, m:m + 1] = v[:, None]
        if 1 <= m <= _K:
            idx_ref[:, m - 1:m] = (j + (vmi & 3) * q)[:, None]
        if m < _NSEL - 1:
            mask = cols == j[:, None]
            m1 = jnp.where(mask, m2, m1)
            m2 = jnp.where(mask, m3, m2)
            m3 = jnp.where(mask, m4, m3)
            m4 = jnp.where(mask, 2.0, m4)


def _stage1(h, l):
    b, c = h.shape
    r = _pick_rows(b)
    grid = (b // r,)
    return pl.pallas_call(
        _stage1_body,
        grid=grid,
        in_specs=[
            pl.BlockSpec((b, c), lambda i: (0, 0)),
            pl.BlockSpec((r, c), lambda i: (i, 0)),
            pl.BlockSpec((r, b), lambda i: (i, 0)),
        ],
        out_specs=[
            pl.BlockSpec((r, _NSEL), lambda i: (i, 0)),
            pl.BlockSpec((r, _K), lambda i: (i, 0)),
            pl.BlockSpec((r, 2), lambda i: (i, 0)),
        ],
        out_shape=[
            jax.ShapeDtypeStruct((b, _NSEL), jnp.float32),
            jax.ShapeDtypeStruct((b, _K), jnp.int32),
            jax.ShapeDtypeStruct((b, 2), jnp.float32),
        ],
    )(h, h, l)


def _stats_body(f_ref, fb_ref, mnmx_ref):
    f = f_ref[...]
    fb = fb_ref[...]
    g = jnp.dot(fb, f.T, preferred_element_type=jnp.float32)
    mnmx_ref[:, 0:1] = jnp.min(g, axis=1, keepdims=True)
    mnmx_ref[:, 1:2] = jnp.max(g, axis=1, keepdims=True)


def _stats(f):
    b, c = f.shape
    r = _pick_rows(b)
    return pl.pallas_call(
        _stats_body,
        grid=(b // r,),
        in_specs=[
            pl.BlockSpec((b, c), lambda i: (0, 0)),
            pl.BlockSpec((r, c), lambda i: (i, 0)),
        ],
        out_specs=pl.BlockSpec((r, 2), lambda i: (i, 0)),
        out_shape=jax.ShapeDtypeStruct((b, 2), jnp.float32),
    )(f, f)


_KW = 24


def _sc_apply_body(shared_tables, c, rw, refs):
    if shared_tables:
        (sum_hbm, idx_hbm, c0_hbm, c1_hbm, c2_hbm, out_hbm,
         idxv, c0v, c1v, c2v, obuf, gsum, sems) = refs
        dot_hbm, gdot = sum_hbm, gsum
    else:
        (sum_hbm, dot_hbm, idx_hbm, c0_hbm, c1_hbm, c2_hbm, out_hbm,
         idxv, c0v, c1v, c2v, obuf, gsum, gdot, sems) = refs
    nchunk = c // _LANES
    wid = lax.axis_index("s") * 2 + lax.axis_index("c")
    base = wid * rw
    pltpu.sync_copy(idx_hbm.at[pl.ds(base, rw)], idxv)
    pltpu.sync_copy(c0_hbm.at[pl.ds(base, rw)], c0v)
    pltpu.sync_copy(c1_hbm.at[pl.ds(base, rw)], c1v)
    pltpu.sync_copy(c2_hbm.at[pl.ds(base, rw)], c2v)

    iot = lax.iota(jnp.int32, _LANES)

    def fire(r, slot):
        pltpu.async_copy(sum_hbm.at[idxv.at[r]], gsum.at[slot], sems[slot])
        if not shared_tables:
            pltpu.async_copy(dot_hbm.at[idxv.at[r]], gdot.at[slot],
                             sems[2 + slot])

    def drain(r, slot):
        pltpu.make_async_copy(sum_hbm.at[idxv.at[r]], gsum.at[slot],
                              sems[slot]).wait()
        if not shared_tables:
            pltpu.make_async_copy(dot_hbm.at[idxv.at[r]], gdot.at[slot],
                                  sems[2 + slot]).wait()

    def compute_row(r, slot):
        own_d = [gdot[slot, 0, pl.ds(ch * _LANES, _LANES)]
                 for ch in range(nchunk)]
        dots = jnp.zeros((_LANES,), jnp.float32)
        for k in range(_K):
            acc = own_d[0] * gdot[slot, 1 + k, pl.ds(0, _LANES)]
            for ch in range(1, nchunk):
                acc = acc + own_d[ch] * gdot[
                    slot, 1 + k, pl.ds(ch * _LANES, _LANES)]
            dots = jnp.where(iot == k, jnp.sum(acc), dots)
        c0row = c0v[r, :]
        c1row = c1v[r, :]
        c2row = c2v[r, :]
        a = jnp.maximum(c0row + c1row * (dots - c2row), 0.0)
        oacc = [jnp.zeros((_LANES,), jnp.float32) for _ in range(nchunk)]
        for k in range(_K):
            ab = jnp.sum(jnp.where(iot == k, a, 0.0))
            for ch in range(nchunk):
                vec = gsum[slot, 1 + k, pl.ds(ch * _LANES, _LANES)]
                oacc[ch] = oacc[ch] + ab * vec
        @pl.when(r >= 2)
        def _():
            pltpu.make_async_copy(obuf.at[slot], out_hbm.at[base + r - 2],
                                  sems[-2 + slot]).wait()
        for ch in range(nchunk):
            own = gsum[slot, 0, pl.ds(ch * _LANES, _LANES)]
            res = _EPSILON * oacc[ch] + (1.0 - _EPSILON) * own
            obuf[slot, pl.ds(ch * _LANES, _LANES)] = res
        pltpu.async_copy(obuf.at[slot], out_hbm.at[base + r],
                         sems[-2 + slot])

    fire(0, 0)

    def step(i, carry):
        r0 = 2 * i
        fire(r0 + 1, 1)
        drain(r0, 0)
        compute_row(r0, 0)
        fire(jnp.minimum(r0 + 2, rw - 1), 0)
        drain(r0 + 1, 1)
        compute_row(r0 + 1, 1)
        return carry

    lax.fori_loop(0, rw // 2, step, 0)
    drain(rw - 1, 0)
    for slot in range(2):
        pltpu.make_async_copy(obuf.at[slot],
                              out_hbm.at[base + rw - 2 + slot],
                              sems[-2 + slot]).wait()


def _sc_apply(sum_tab, dot_tab, idx, c0, c1, c2, shared_tables):
    bp, c = sum_tab.shape
    rw = bp // _NW
    mesh = plsc.VectorSubcoreMesh(core_axis_name="c", subcore_axis_name="s")
    scratch = [
        pltpu.VMEM((rw, _KW), jnp.int32),
        pltpu.VMEM((rw, _K), jnp.float32),
        pltpu.VMEM((rw, _K), jnp.float32),
        pltpu.VMEM((rw, _K), jnp.float32),
        pltpu.VMEM((2, c), jnp.float32),
        pltpu.VMEM((2, _KW, c), jnp.float32),
    ]
    if not shared_tables:
        scratch.append(pltpu.VMEM((2, _KW, c), jnp.float32))
    nsem = 4 if shared_tables else 6
    for _ in range(nsem):
        scratch.append(pltpu.SemaphoreType.DMA)

    def body(*refs):
        if shared_tables:
            (sum_hbm, idx_hbm, c0_hbm, c1_hbm, c2_hbm, out_hbm,
             idxv, c0v, c1v, c2v, obuf, gsum, s0, s1, o0, o1) = refs
            _sc_apply_body(True, c, rw,
                           (sum_hbm, idx_hbm, c0_hbm, c1_hbm, c2_hbm,
                            out_hbm, idxv, c0v, c1v, c2v, obuf, gsum,
                            [s0, s1, o0, o1]))
        else:
            (sum_hbm, dot_hbm, idx_hbm, c0_hbm, c1_hbm, c2_hbm, out_hbm,
             idxv, c0v, c1v, c2v, obuf, gsum, gdot,
             s0, s1, s2, s3, o0, o1) = refs
            _sc_apply_body(False, c, rw,
                           (sum_hbm, dot_hbm, idx_hbm, c0_hbm, c1_hbm,
                            c2_hbm, out_hbm, idxv, c0v, c1v, c2v, obuf,
                            gsum, gdot, [s0, s1, s2, s3, o0, o1]))

    kern = functools.partial(
        pl.kernel, mesh=mesh,
        out_type=jax.ShapeDtypeStruct((bp, c), jnp.float32),
        scratch_types=scratch,
        compiler_params=pltpu.CompilerParams(
            needs_layout_passes=False, use_tc_tiling_on_sc=False),
    )(body)
    if shared_tables:
        return kern(sum_tab, idx, c0, c1, c2)
    return kern(sum_tab, dot_tab, idx, c0, c1, c2)


def _pad_rows(a, bp):
    pad = [(0, bp - a.shape[0])] + [(0, 0)] * (a.ndim - 1)
    return jnp.pad(a, pad)


def kernel(x, L):
    h = x[0]
    b, c = h.shape
    bp = ((b + 8 * _NW - 1) // (8 * _NW)) * (8 * _NW)
    vals, idx, mnmx0 = _stage1(h, L)
    dval = vals[:, 1:_K + 1]
    dk = vals[:, _K + 1]
    gamma = jnp.mean(0.5 * (_K * dk - jnp.sum(dval, axis=1)))
    inv2g = 1.0 / (2.0 * gamma + 1e-8)
    eta = (1.0 / _K) * (1.0 + jnp.sum(dval, axis=1) * inv2g)

    c0 = eta[:, None] - inv2g * dval
    hp = _pad_rows(h, bp)
    own = jnp.arange(b, dtype=jnp.int32)[:, None]
    idxw = jnp.concatenate(
        [own, idx, jnp.broadcast_to(own, (b, _KW - _K - 1))], axis=1)
    idxp = _pad_rows(idxw, bp)
    c0p = _pad_rows(c0, bp)

    def consts(mnmx):
        c1 = inv2g * _LAM / (mnmx[:, 1] - mnmx[:, 0] + 1e-8)
        c1k = jnp.broadcast_to(c1[:, None], (b, _K))
        c2k = jnp.broadcast_to(mnmx[:, 0:1], (b, _K))
        return _pad_rows(c1k, bp), _pad_rows(c2k, bp)

    c1p, c2p = consts(mnmx0)
    fm1p = _sc_apply(hp, hp, idxp, c0p, c1p, c2p, shared_tables=True)

    mnmx1 = _stats(fm1p[:b])
    c1p, c2p = consts(mnmx1)
    fm2p = _sc_apply(hp, fm1p, idxp, c0p, c1p, c2p, shared_tables=False)
    return fm2p[:b][None, :, :]


# speedup vs baseline: 19.2648x; 1.0146x over previous
import functools

import jax
import jax.numpy as jnp
from jax import lax
from jax.experimental import pallas as pl
from jax.experimental.pallas import tpu as pltpu
from jax.experimental.pallas import tpu_sc as plsc

_K = 16
_EPSILON = 0.5
_LAM = 0.1
_BETA = 0.1
_NSEL = _K + 2

_NW = 32
_LANES = 16


def _pick_rows(b):
    for r in (256, 200, 128, 80, 64, 40, 32, 16, 8):
        if b % r == 0:
            return r
    return b


def _stage1_body(h_ref, hb_ref, l_ref, vals_ref, idx_ref, mnmx_ref):
    h = h_ref[...]
    hb = hb_ref[...]
    lb = l_ref[...]
    g = jnp.dot(hb, h.T, preferred_element_type=jnp.float32)
    xx_b = jnp.sum(hb * hb, axis=1, keepdims=True)
    yy = jnp.sum(h * h, axis=1)[None, :]
    mn_g = jnp.min(g, axis=1, keepdims=True)
    mx_g = jnp.max(g, axis=1, keepdims=True)
    mnmx_ref[:, 0:1] = mn_g
    mnmx_ref[:, 1:2] = mx_g
    a_row = 1e-5 / (mx_g - mn_g + 1e-8)
    dist = jnp.sqrt(jnp.clip(xx_b + yy - 2.0 * g, 1e-12, None))
    d1 = dist - 2.0 * _BETA * lb - a_row * g
    mn1 = jnp.min(d1, axis=1, keepdims=True)
    mx1 = jnp.max(d1, axis=1, keepdims=True)
    d = (d1 - mn1) / (mx1 - mn1 + 1e-8)
    q = d.shape[1] // 4
    m1, m2, m3, m4 = (
        jax.lax.bitcast_convert_type(
            (jax.lax.bitcast_convert_type(
                d[:, s * q:(s + 1) * q], jnp.int32) & ~3) | s,
            jnp.float32)
        for s in range(4))
    m1, m2 = jnp.minimum(m1, m2), jnp.maximum(m1, m2)
    m3, m4 = jnp.minimum(m3, m4), jnp.maximum(m3, m4)
    m1, m3 = jnp.minimum(m1, m3), jnp.maximum(m1, m3)
    m2, m4 = jnp.minimum(m2, m4), jnp.maximum(m2, m4)
    m2, m3 = jnp.minimum(m2, m3), jnp.maximum(m2, m3)
    cols = jax.lax.broadcasted_iota(jnp.int32, m1.shape, 1)
    for m in range(_NSEL):
        vmf = jnp.min(m1, axis=1)
        j = jnp.argmin(m1, axis=1).astype(jnp.int32)
        vmi = jax.lax.bitcast_convert_type(vmf, jnp.int32)
        v = jax.lax.bitcast_convert_type(vmi & ~3, jnp.float32)
        vals_ref[:, m:m + 1] = v[:, None]
        if 1 <= m <= _K:
            idx_ref[:, m - 1:m] = (j + (vmi & 3) * q)[:, None]
        if m < _NSEL - 1:
            mask = cols == j[:, None]
            m1 = jnp.where(mask, m2, m1)
            m2 = jnp.where(mask, m3, m2)
            m3 = jnp.where(mask, m4, m3)
            m4 = jnp.where(mask, 2.0, m4)


def _stage1(h, l):
    b, c = h.shape
    r = _pick_rows(b)
    grid = (b // r,)
    return pl.pallas_call(
        _stage1_body,
        grid=grid,
        in_specs=[
            pl.BlockSpec((b, c), lambda i: (0, 0)),
            pl.BlockSpec((r, c), lambda i: (i, 0)),
            pl.BlockSpec((r, b), lambda i: (i, 0)),
        ],
        out_specs=[
            pl.BlockSpec((r, _NSEL), lambda i: (i, 0)),
            pl.BlockSpec((r, _K), lambda i: (i, 0)),
            pl.BlockSpec((r, 2), lambda i: (i, 0)),
        ],
        out_shape=[
            jax.ShapeDtypeStruct((b, _NSEL), jnp.float32),
            jax.ShapeDtypeStruct((b, _K), jnp.int32),
            jax.ShapeDtypeStruct((b, 2), jnp.float32),
        ],
    )(h, h, l)


def _stats_body(f_ref, fb_ref, mnmx_ref):
    f = f_ref[...]
    fb = fb_ref[...]
    g = jnp.dot(fb, f.T, preferred_element_type=jnp.float32)
    mnmx_ref[:, 0:1] = jnp.min(g, axis=1, keepdims=True)
    mnmx_ref[:, 1:2] = jnp.max(g, axis=1, keepdims=True)


def _stats(f):
    b, c = f.shape
    r = _pick_rows(b)
    return pl.pallas_call(
        _stats_body,
        grid=(b // r,),
        in_specs=[
            pl.BlockSpec((b, c), lambda i: (0, 0)),
            pl.BlockSpec((r, c), lambda i: (i, 0)),
        ],
        out_specs=pl.BlockSpec((r, 2), lambda i: (i, 0)),
        out_shape=jax.ShapeDtypeStruct((b, 2), jnp.float32),
    )(f, f)


_KW = 24


def _sc_apply_body(shared_tables, c, rw, refs):
    if shared_tables:
        (sum_hbm, idx_hbm, c0_hbm, c1_hbm, c2_hbm, out_hbm,
         idxv, c0v, c1v, c2v, obuf, gsum, av, sems) = refs
        dot_hbm, gdot = sum_hbm, gsum
    else:
        (sum_hbm, dot_hbm, idx_hbm, c0_hbm, c1_hbm, c2_hbm, out_hbm,
         idxv, c0v, c1v, c2v, obuf, gsum, gdot, av, sems) = refs
    nchunk = c // _LANES
    wid = lax.axis_index("s") * 2 + lax.axis_index("c")
    base = wid * rw
    pltpu.sync_copy(idx_hbm.at[pl.ds(base, rw)], idxv)
    pltpu.sync_copy(c0_hbm.at[pl.ds(base, rw)], c0v)
    pltpu.sync_copy(c1_hbm.at[pl.ds(base, rw)], c1v)
    pltpu.sync_copy(c2_hbm.at[pl.ds(base, rw)], c2v)

    iot = lax.iota(jnp.int32, _LANES)

    def fire(r, slot):
        pltpu.async_copy(sum_hbm.at[idxv.at[r]], gsum.at[slot], sems[slot])
        if not shared_tables:
            pltpu.async_copy(dot_hbm.at[idxv.at[r]], gdot.at[slot],
                             sems[2 + slot])

    def drain(r, slot):
        pltpu.make_async_copy(sum_hbm.at[idxv.at[r]], gsum.at[slot],
                              sems[slot]).wait()
        if not shared_tables:
            pltpu.make_async_copy(dot_hbm.at[idxv.at[r]], gdot.at[slot],
                                  sems[2 + slot]).wait()

    def compute_row(r, slot):
        own_d = [gdot[slot, 0, pl.ds(ch * _LANES, _LANES)]
                 for ch in range(nchunk)]
        dots = jnp.zeros((_LANES,), jnp.float32)
        for k in range(_K):
            acc = own_d[0] * gdot[slot, 1 + k, pl.ds(0, _LANES)]
            for ch in range(1, nchunk):
                acc = acc + own_d[ch] * gdot[
                    slot, 1 + k, pl.ds(ch * _LANES, _LANES)]
            dots = jnp.where(iot == k, jnp.sum(acc), dots)
        c0row = c0v[r, :]
        c1row = c1v[r, :]
        c2row = c2v[r, :]
        a = jnp.maximum(c0row + c1row * (dots - c2row), 0.0)
        oacc = [jnp.zeros((_LANES,), jnp.float32) for _ in range(nchunk)]
        for k in range(_K):
            ab = jnp.sum(jnp.where(iot == k, a, 0.0))
            for ch in range(nchunk):
                vec = gsum[slot, 1 + k, pl.ds(ch * _LANES, _LANES)]
                oacc[ch] = oacc[ch] + ab * vec
        @pl.when(r >= 2)
        def _():
            pltpu.make_async_copy(obuf.at[slot], out_hbm.at[base + r - 2],
                                  sems[-2 + slot]).wait()
        for ch in range(nchunk):
            own = gsum[slot, 0, pl.ds(ch * _LANES, _LANES)]
            res = _EPSILON * oacc[ch] + (1.0 - _EPSILON) * own
            obuf[slot, pl.ds(ch * _LANES, _LANES)] = res
        pltpu.async_copy(obuf.at[slot], out_hbm.at[base + r],
                         sems[-2 + slot])

    fire(0, 0)

    def step(i, carry):
        r0 = 2 * i
        fire(r0 + 1, 1)
        drain(r0, 0)
        compute_row(r0, 0)
        fire(jnp.minimum(r0 + 2, rw - 1), 0)
        drain(r0 + 1, 1)
        compute_row(r0 + 1, 1)
        return carry

    lax.fori_loop(0, rw // 2, step, 0)
    drain(rw - 1, 0)
    for slot in range(2):
        pltpu.make_async_copy(obuf.at[slot],
                              out_hbm.at[base + rw - 2 + slot],
                              sems[-2 + slot]).wait()


def _sc_apply(sum_tab, dot_tab, idx, c0, c1, c2, shared_tables):
    bp, c = sum_tab.shape
    rw = bp // _NW
    mesh = plsc.VectorSubcoreMesh(core_axis_name="c", subcore_axis_name="s")
    scratch = [
        pltpu.VMEM((rw, _KW), jnp.int32),
        pltpu.VMEM((rw, _K), jnp.float32),
        pltpu.VMEM((rw, _K), jnp.float32),
        pltpu.VMEM((rw, _K), jnp.float32),
        pltpu.VMEM((2, c), jnp.float32),
        pltpu.VMEM((2, _KW, c), jnp.float32),
    ]
    if not shared_tables:
        scratch.append(pltpu.VMEM((2, _KW, c), jnp.float32))
    scratch.append(pltpu.VMEM((_LANES,), jnp.float32))
    nsem = 4 if shared_tables else 6
    for _ in range(nsem):
        scratch.append(pltpu.SemaphoreType.DMA)

    def body(*refs):
        if shared_tables:
            (sum_hbm, idx_hbm, c0_hbm, c1_hbm, c2_hbm, out_hbm,
             idxv, c0v, c1v, c2v, obuf, gsum, av, s0, s1, o0, o1) = refs
            _sc_apply_body(True, c, rw,
                           (sum_hbm, idx_hbm, c0_hbm, c1_hbm, c2_hbm,
                            out_hbm, idxv, c0v, c1v, c2v, obuf, gsum, av,
                            [s0, s1, o0, o1]))
        else:
            (sum_hbm, dot_hbm, idx_hbm, c0_hbm, c1_hbm, c2_hbm, out_hbm,
             idxv, c0v, c1v, c2v, obuf, gsum, gdot, av,
             s0, s1, s2, s3, o0, o1) = refs
            _sc_apply_body(False, c, rw,
                           (sum_hbm, dot_hbm, idx_hbm, c0_hbm, c1_hbm,
                            c2_hbm, out_hbm, idxv, c0v, c1v, c2v, obuf,
                            gsum, gdot, av, [s0, s1, s2, s3, o0, o1]))

    kern = functools.partial(
        pl.kernel, mesh=mesh,
        out_type=jax.ShapeDtypeStruct((bp, c), jnp.float32),
        scratch_types=scratch,
        compiler_params=pltpu.CompilerParams(
            needs_layout_passes=False, use_tc_tiling_on_sc=False),
    )(body)
    if shared_tables:
        return kern(sum_tab, idx, c0, c1, c2)
    return kern(sum_tab, dot_tab, idx, c0, c1, c2)


def _pad_rows(a, bp):
    pad = [(0, bp - a.shape[0])] + [(0, 0)] * (a.ndim - 1)
    return jnp.pad(a, pad)


def kernel(x, L):
    h = x[0]
    b, c = h.shape
    bp = ((b + 8 * _NW - 1) // (8 * _NW)) * (8 * _NW)
    vals, idx, mnmx0 = _stage1(h, L)
    dval = vals[:, 1:_K + 1]
    dk = vals[:, _K + 1]
    gamma = jnp.mean(0.5 * (_K * dk - jnp.sum(dval, axis=1)))
    inv2g = 1.0 / (2.0 * gamma + 1e-8)
    eta = (1.0 / _K) * (1.0 + jnp.sum(dval, axis=1) * inv2g)

    c0 = eta[:, None] - inv2g * dval
    hp = _pad_rows(h, bp)
    own = jnp.arange(b, dtype=jnp.int32)[:, None]
    idxw = jnp.concatenate(
        [own, idx, jnp.broadcast_to(own, (b, _KW - _K - 1))], axis=1)
    idxp = _pad_rows(idxw, bp)
    c0p = _pad_rows(c0, bp)

    def consts(mnmx):
        c1 = inv2g * _LAM / (mnmx[:, 1] - mnmx[:, 0] + 1e-8)
        c1k = jnp.broadcast_to(c1[:, None], (b, _K))
        c2k = jnp.broadcast_to(mnmx[:, 0:1], (b, _K))
        return _pad_rows(c1k, bp), _pad_rows(c2k, bp)

    c1p, c2p = consts(mnmx0)
    fm1p = _sc_apply(hp, hp, idxp, c0p, c1p, c2p, shared_tables=True)

    mnmx1 = _stats(fm1p[:b])
    c1p, c2p = consts(mnmx1)
    fm2p = _sc_apply(hp, fm1p, idxp, c0p, c1p, c2p, shared_tables=False)
    return fm2p[:b][None, :, :]


# speedup vs baseline: 20.1608x; 1.0465x over previous
import functools

import jax
import jax.numpy as jnp
from jax import lax
from jax.experimental import pallas as pl
from jax.experimental.pallas import tpu as pltpu
from jax.experimental.pallas import tpu_sc as plsc

_K = 16
_EPSILON = 0.5
_LAM = 0.1
_BETA = 0.1
_NSEL = _K + 2

_NW = 32
_LANES = 16


def _pick_rows(b):
    for r in (256, 200, 128, 80, 64, 40, 32, 16, 8):
        if b % r == 0:
            return r
    return b


def _stage1_body(h_ref, hb_ref, l_ref, vals_ref, idx_ref, mnmx_ref):
    h = h_ref[...]
    hb = hb_ref[...]
    lb = l_ref[...]
    g = jnp.dot(hb, h.T, preferred_element_type=jnp.float32)
    xx_b = jnp.sum(hb * hb, axis=1, keepdims=True)
    yy = jnp.sum(h * h, axis=1)[None, :]
    mn_g = jnp.min(g, axis=1, keepdims=True)
    mx_g = jnp.max(g, axis=1, keepdims=True)
    mnmx_ref[:, 0:1] = mn_g
    mnmx_ref[:, 1:2] = mx_g
    a_row = 1e-5 / (mx_g - mn_g + 1e-8)
    dist = jnp.sqrt(jnp.clip(xx_b + yy - 2.0 * g, 1e-12, None))
    d1 = dist - 2.0 * _BETA * lb - a_row * g
    mn1 = jnp.min(d1, axis=1, keepdims=True)
    mx1 = jnp.max(d1, axis=1, keepdims=True)
    d = (d1 - mn1) / (mx1 - mn1 + 1e-8)
    nf = 8
    q = d.shape[1] // nf
    ms = [
        jax.lax.bitcast_convert_type(
            (jax.lax.bitcast_convert_type(
                d[:, s * q:(s + 1) * q], jnp.int32) & ~(nf - 1)) | s,
            jnp.float32)
        for s in range(nf)]

    def ce(i, jj):
        ms[i], ms[jj] = jnp.minimum(ms[i], ms[jj]), jnp.maximum(
            ms[i], ms[jj])

    for pair in ((0, 1), (2, 3), (4, 5), (6, 7),
                 (0, 2), (1, 3), (4, 6), (5, 7),
                 (1, 2), (5, 6),
                 (0, 4), (1, 5), (2, 6), (3, 7),
                 (2, 4), (3, 5),
                 (1, 2), (3, 4), (5, 6)):
        ce(*pair)
    cols = jax.lax.broadcasted_iota(jnp.int32, ms[0].shape, 1)
    for m in range(_NSEL):
        vmf = jnp.min(ms[0], axis=1)
        j = jnp.argmin(ms[0], axis=1).astype(jnp.int32)
        vmi = jax.lax.bitcast_convert_type(vmf, jnp.int32)
        v = jax.lax.bitcast_convert_type(vmi & ~(nf - 1), jnp.float32)
        vals_ref[:, m:m + 1] = v[:, None]
        if 1 <= m <= _K:
            idx_ref[:, m - 1:m] = (j + (vmi & (nf - 1)) * q)[:, None]
        if m < _NSEL - 1:
            mask = cols == j[:, None]
            for i in range(nf - 1):
                ms[i] = jnp.where(mask, ms[i + 1], ms[i])
            ms[nf - 1] = jnp.where(mask, 2.0, ms[nf - 1])


def _stage1(h, l):
    b, c = h.shape
    r = _pick_rows(b)
    grid = (b // r,)
    return pl.pallas_call(
        _stage1_body,
        grid=grid,
        in_specs=[
            pl.BlockSpec((b, c), lambda i: (0, 0)),
            pl.BlockSpec((r, c), lambda i: (i, 0)),
            pl.BlockSpec((r, b), lambda i: (i, 0)),
        ],
        out_specs=[
            pl.BlockSpec((r, _NSEL), lambda i: (i, 0)),
            pl.BlockSpec((r, _K), lambda i: (i, 0)),
            pl.BlockSpec((r, 2), lambda i: (i, 0)),
        ],
        out_shape=[
            jax.ShapeDtypeStruct((b, _NSEL), jnp.float32),
            jax.ShapeDtypeStruct((b, _K), jnp.int32),
            jax.ShapeDtypeStruct((b, 2), jnp.float32),
        ],
    )(h, h, l)


def _stats_body(f_ref, fb_ref, mnmx_ref):
    f = f_ref[...]
    fb = fb_ref[...]
    g = jnp.dot(fb, f.T, preferred_element_type=jnp.float32)
    mnmx_ref[:, 0:1] = jnp.min(g, axis=1, keepdims=True)
    mnmx_ref[:, 1:2] = jnp.max(g, axis=1, keepdims=True)


def _stats(f):
    b, c = f.shape
    r = _pick_rows(b)
    return pl.pallas_call(
        _stats_body,
        grid=(b // r,),
        in_specs=[
            pl.BlockSpec((b, c), lambda i: (0, 0)),
            pl.BlockSpec((r, c), lambda i: (i, 0)),
        ],
        out_specs=pl.BlockSpec((r, 2), lambda i: (i, 0)),
        out_shape=jax.ShapeDtypeStruct((b, 2), jnp.float32),
    )(f, f)


_KW = 24


def _sc_apply_body(shared_tables, c, rw, refs):
    if shared_tables:
        (sum_hbm, idx_hbm, c0_hbm, c1_hbm, c2_hbm, out_hbm,
         idxv, c0v, c1v, c2v, obuf, gsum, av, sems) = refs
        dot_hbm, gdot = sum_hbm, gsum
    else:
        (sum_hbm, dot_hbm, idx_hbm, c0_hbm, c1_hbm, c2_hbm, out_hbm,
         idxv, c0v, c1v, c2v, obuf, gsum, gdot, av, sems) = refs
    nchunk = c // _LANES
    wid = lax.axis_index("s") * 2 + lax.axis_index("c")
    base = wid * rw
    pltpu.sync_copy(idx_hbm.at[pl.ds(base, rw)], idxv)
    pltpu.sync_copy(c0_hbm.at[pl.ds(base, rw)], c0v)
    pltpu.sync_copy(c1_hbm.at[pl.ds(base, rw)], c1v)
    pltpu.sync_copy(c2_hbm.at[pl.ds(base, rw)], c2v)

    iot = lax.iota(jnp.int32, _LANES)

    def fire(r, slot):
        pltpu.async_copy(sum_hbm.at[idxv.at[r]], gsum.at[slot], sems[slot])
        if not shared_tables:
            pltpu.async_copy(dot_hbm.at[idxv.at[r]], gdot.at[slot],
                             sems[2 + slot])

    def drain(r, slot):
        pltpu.make_async_copy(sum_hbm.at[idxv.at[r]], gsum.at[slot],
                              sems[slot]).wait()
        if not shared_tables:
            pltpu.make_async_copy(dot_hbm.at[idxv.at[r]], gdot.at[slot],
                                  sems[2 + slot]).wait()

    def compute_row(r, slot):
        own_d = [gdot[slot, 0, pl.ds(ch * _LANES, _LANES)]
                 for ch in range(nchunk)]
        dots = jnp.zeros((_LANES,), jnp.float32)
        for k in range(_K):
            acc = own_d[0] * gdot[slot, 1 + k, pl.ds(0, _LANES)]
            for ch in range(1, nchunk):
                acc = acc + own_d[ch] * gdot[
                    slot, 1 + k, pl.ds(ch * _LANES, _LANES)]
            dots = jnp.where(iot == k, jnp.sum(acc), dots)
        c0row = c0v[r, :]
        c1row = c1v[r, :]
        c2row = c2v[r, :]
        a = jnp.maximum(c0row + c1row * (dots - c2row), 0.0)
        oacc = [jnp.zeros((_LANES,), jnp.float32) for _ in range(nchunk)]
        for k in range(_K):
            ab = jnp.sum(jnp.where(iot == k, a, 0.0))
            for ch in range(nchunk):
                vec = gsum[slot, 1 + k, pl.ds(ch * _LANES, _LANES)]
                oacc[ch] = oacc[ch] + ab * vec
        @pl.when(r >= 2)
        def _():
            pltpu.make_async_copy(obuf.at[slot], out_hbm.at[base + r - 2],
                                  sems[-2 + slot]).wait()
        for ch in range(nchunk):
            own = gsum[slot, 0, pl.ds(ch * _LANES, _LANES)]
            res = _EPSILON * oacc[ch] + (1.0 - _EPSILON) * own
            obuf[slot, pl.ds(ch * _LANES, _LANES)] = res
        pltpu.async_copy(obuf.at[slot], out_hbm.at[base + r],
                         sems[-2 + slot])

    fire(0, 0)

    def step(i, carry):
        r0 = 2 * i
        fire(r0 + 1, 1)
        drain(r0, 0)
        compute_row(r0, 0)
        fire(jnp.minimum(r0 + 2, rw - 1), 0)
        drain(r0 + 1, 1)
        compute_row(r0 + 1, 1)
        return carry

    lax.fori_loop(0, rw // 2, step, 0)
    drain(rw - 1, 0)
    for slot in range(2):
        pltpu.make_async_copy(obuf.at[slot],
                              out_hbm.at[base + rw - 2 + slot],
                              sems[-2 + slot]).wait()


def _sc_apply(sum_tab, dot_tab, idx, c0, c1, c2, shared_tables):
    bp, c = sum_tab.shape
    rw = bp // _NW
    mesh = plsc.VectorSubcoreMesh(core_axis_name="c", subcore_axis_name="s")
    scratch = [
        pltpu.VMEM((rw, _KW), jnp.int32),
        pltpu.VMEM((rw, _K), jnp.float32),
        pltpu.VMEM((rw, _K), jnp.float32),
        pltpu.VMEM((rw, _K), jnp.float32),
        pltpu.VMEM((2, c), jnp.float32),
        pltpu.VMEM((2, _KW, c), jnp.float32),
    ]
    if not shared_tables:
        scratch.append(pltpu.VMEM((2, _KW, c), jnp.float32))
    scratch.append(pltpu.VMEM((_LANES,), jnp.float32))
    nsem = 4 if shared_tables else 6
    for _ in range(nsem):
        scratch.append(pltpu.SemaphoreType.DMA)

    def body(*refs):
        if shared_tables:
            (sum_hbm, idx_hbm, c0_hbm, c1_hbm, c2_hbm, out_hbm,
             idxv, c0v, c1v, c2v, obuf, gsum, av, s0, s1, o0, o1) = refs
            _sc_apply_body(True, c, rw,
                           (sum_hbm, idx_hbm, c0_hbm, c1_hbm, c2_hbm,
                            out_hbm, idxv, c0v, c1v, c2v, obuf, gsum, av,
                            [s0, s1, o0, o1]))
        else:
            (sum_hbm, dot_hbm, idx_hbm, c0_hbm, c1_hbm, c2_hbm, out_hbm,
             idxv, c0v, c1v, c2v, obuf, gsum, gdot, av,
             s0, s1, s2, s3, o0, o1) = refs
            _sc_apply_body(False, c, rw,
                           (sum_hbm, dot_hbm, idx_hbm, c0_hbm, c1_hbm,
                            c2_hbm, out_hbm, idxv, c0v, c1v, c2v, obuf,
                            gsum, gdot, av, [s0, s1, s2, s3, o0, o1]))

    kern = functools.partial(
        pl.kernel, mesh=mesh,
        out_type=jax.ShapeDtypeStruct((bp, c), jnp.float32),
        scratch_types=scratch,
        compiler_params=pltpu.CompilerParams(
            needs_layout_passes=False, use_tc_tiling_on_sc=False),
    )(body)
    if shared_tables:
        return kern(sum_tab, idx, c0, c1, c2)
    return kern(sum_tab, dot_tab, idx, c0, c1, c2)


def _pad_rows(a, bp):
    pad = [(0, bp - a.shape[0])] + [(0, 0)] * (a.ndim - 1)
    return jnp.pad(a, pad)


def kernel(x, L):
    h = x[0]
    b, c = h.shape
    bp = ((b + 8 * _NW - 1) // (8 * _NW)) * (8 * _NW)
    vals, idx, mnmx0 = _stage1(h, L)
    dval = vals[:, 1:_K + 1]
    dk = vals[:, _K + 1]
    gamma = jnp.mean(0.5 * (_K * dk - jnp.sum(dval, axis=1)))
    inv2g = 1.0 / (2.0 * gamma + 1e-8)
    eta = (1.0 / _K) * (1.0 + jnp.sum(dval, axis=1) * inv2g)

    c0 = eta[:, None] - inv2g * dval
    hp = _pad_rows(h, bp)
    own = jnp.arange(b, dtype=jnp.int32)[:, None]
    idxw = jnp.concatenate(
        [own, idx, jnp.broadcast_to(own, (b, _KW - _K - 1))], axis=1)
    idxp = _pad_rows(idxw, bp)
    c0p = _pad_rows(c0, bp)

    def consts(mnmx):
        c1 = inv2g * _LAM / (mnmx[:, 1] - mnmx[:, 0] + 1e-8)
        c1k = jnp.broadcast_to(c1[:, None], (b, _K))
        c2k = jnp.broadcast_to(mnmx[:, 0:1], (b, _K))
        return _pad_rows(c1k, bp), _pad_rows(c2k, bp)

    c1p, c2p = consts(mnmx0)
    fm1p = _sc_apply(hp, hp, idxp, c0p, c1p, c2p, shared_tables=True)

    mnmx1 = _stats(fm1p[:b])
    c1p, c2p = consts(mnmx1)
    fm2p = _sc_apply(hp, fm1p, idxp, c0p, c1p, c2p, shared_tables=False)
    return fm2p[:b][None, :, :]


# speedup vs baseline: 20.1766x; 1.0008x over previous
import functools

import jax
import jax.numpy as jnp
from jax import lax
from jax.experimental import pallas as pl
from jax.experimental.pallas import tpu as pltpu
from jax.experimental.pallas import tpu_sc as plsc

_K = 16
_EPSILON = 0.5
_LAM = 0.1
_BETA = 0.1
_NSEL = _K + 2

_NW = 32
_LANES = 16


def _pick_rows(b):
    for r in (256, 200, 128, 80, 64, 40, 32, 16, 8):
        if b % r == 0:
            return r
    return b


def _stage1_body(h_ref, hb_ref, l_ref, vals_ref, idx_ref, mnmx_ref):
    h = h_ref[...]
    hb = hb_ref[...]
    lb = l_ref[...]
    g = jnp.dot(hb, h.T, preferred_element_type=jnp.float32)
    xx_b = jnp.sum(hb * hb, axis=1, keepdims=True)
    yy = jnp.sum(h * h, axis=1)[None, :]
    mn_g = jnp.min(g, axis=1, keepdims=True)
    mx_g = jnp.max(g, axis=1, keepdims=True)
    mnmx_ref[:, 0:1] = mn_g
    mnmx_ref[:, 1:2] = mx_g
    a_row = 1e-5 / (mx_g - mn_g + 1e-8)
    dist = jnp.sqrt(jnp.clip(xx_b + yy - 2.0 * g, 1e-12, None))
    d1 = dist - 2.0 * _BETA * lb - a_row * g
    mn1 = jnp.min(d1, axis=1, keepdims=True)
    mx1 = jnp.max(d1, axis=1, keepdims=True)
    d = (d1 - mn1) / (mx1 - mn1 + 1e-8)
    nf = 8
    q = d.shape[1] // nf
    ms = [
        jax.lax.bitcast_convert_type(
            (jax.lax.bitcast_convert_type(
                d[:, s * q:(s + 1) * q], jnp.int32) & ~(nf - 1)) | s,
            jnp.float32)
        for s in range(nf)]

    def ce(i, jj):
        ms[i], ms[jj] = jnp.minimum(ms[i], ms[jj]), jnp.maximum(
            ms[i], ms[jj])

    for pair in ((0, 1), (2, 3), (4, 5), (6, 7),
                 (0, 2), (1, 3), (4, 6), (5, 7),
                 (1, 2), (5, 6),
                 (0, 4), (1, 5), (2, 6), (3, 7),
                 (2, 4), (3, 5),
                 (1, 2), (3, 4), (5, 6)):
        ce(*pair)
    cols = jax.lax.broadcasted_iota(jnp.int32, ms[0].shape, 1)
    for m in range(_NSEL):
        vmf = jnp.min(ms[0], axis=1)
        j = jnp.argmin(ms[0], axis=1).astype(jnp.int32)
        vmi = jax.lax.bitcast_convert_type(vmf, jnp.int32)
        v = jax.lax.bitcast_convert_type(vmi & ~(nf - 1), jnp.float32)
        vals_ref[:, m:m + 1] = v[:, None]
        if 1 <= m <= _K:
            idx_ref[:, m - 1:m] = (j + (vmi & (nf - 1)) * q)[:, None]
        if m < _NSEL - 1:
            mask = cols == j[:, None]
            for i in range(nf - 1):
                ms[i] = jnp.where(mask, ms[i + 1], ms[i])
            ms[nf - 1] = jnp.where(mask, 2.0, ms[nf - 1])


def _stage1(h, l):
    b, c = h.shape
    r = _pick_rows(b)
    grid = (b // r,)
    return pl.pallas_call(
        _stage1_body,
        grid=grid,
        in_specs=[
            pl.BlockSpec((b, c), lambda i: (0, 0)),
            pl.BlockSpec((r, c), lambda i: (i, 0)),
            pl.BlockSpec((r, b), lambda i: (i, 0)),
        ],
        out_specs=[
            pl.BlockSpec((r, _NSEL), lambda i: (i, 0)),
            pl.BlockSpec((r, _K), lambda i: (i, 0)),
            pl.BlockSpec((r, 2), lambda i: (i, 0)),
        ],
        out_shape=[
            jax.ShapeDtypeStruct((b, _NSEL), jnp.float32),
            jax.ShapeDtypeStruct((b, _K), jnp.int32),
            jax.ShapeDtypeStruct((b, 2), jnp.float32),
        ],
    )(h, h, l)


def _stats_body(f_ref, fb_ref, mnmx_ref):
    f = f_ref[...]
    fb = fb_ref[...]
    g = jnp.dot(fb, f.T, preferred_element_type=jnp.float32)
    mnmx_ref[:, 0:1] = jnp.min(g, axis=1, keepdims=True)
    mnmx_ref[:, 1:2] = jnp.max(g, axis=1, keepdims=True)


def _stats(f):
    b, c = f.shape
    r = _pick_rows(b)
    return pl.pallas_call(
        _stats_body,
        grid=(b // r,),
        in_specs=[
            pl.BlockSpec((b, c), lambda i: (0, 0)),
            pl.BlockSpec((r, c), lambda i: (i, 0)),
        ],
        out_specs=pl.BlockSpec((r, 2), lambda i: (i, 0)),
        out_shape=jax.ShapeDtypeStruct((b, 2), jnp.float32),
    )(f, f)


_KW = 24
_NBUF = 4


def _sc_apply_body(shared_tables, c, rw, refs):
    if shared_tables:
        (sum_hbm, idx_hbm, c0_hbm, c1_hbm, c2_hbm, out_hbm,
         idxv, c0v, c1v, c2v, obuf, gsum, sems) = refs
        dot_hbm, gdot = sum_hbm, gsum
    else:
        (sum_hbm, dot_hbm, idx_hbm, c0_hbm, c1_hbm, c2_hbm, out_hbm,
         idxv, c0v, c1v, c2v, obuf, gsum, gdot, sems) = refs
    nchunk = c // _LANES
    wid = lax.axis_index("s") * 2 + lax.axis_index("c")
    base = wid * rw
    pltpu.sync_copy(idx_hbm.at[pl.ds(base, rw)], idxv)
    pltpu.sync_copy(c0_hbm.at[pl.ds(base, rw)], c0v)
    pltpu.sync_copy(c1_hbm.at[pl.ds(base, rw)], c1v)
    pltpu.sync_copy(c2_hbm.at[pl.ds(base, rw)], c2v)

    iot = lax.iota(jnp.int32, _LANES)

    def fire(r, slot):
        pltpu.async_copy(sum_hbm.at[idxv.at[r]], gsum.at[slot], sems[slot])
        if not shared_tables:
            pltpu.async_copy(dot_hbm.at[idxv.at[r]], gdot.at[slot],
                             sems[_NBUF + slot])

    def drain(r, slot):
        pltpu.make_async_copy(sum_hbm.at[idxv.at[r]], gsum.at[slot],
                              sems[slot]).wait()
        if not shared_tables:
            pltpu.make_async_copy(dot_hbm.at[idxv.at[r]], gdot.at[slot],
                                  sems[_NBUF + slot]).wait()

    def compute_row(r, slot, oslot):
        own_d = [gdot[slot, 0, pl.ds(ch * _LANES, _LANES)]
                 for ch in range(nchunk)]
        dots = jnp.zeros((_LANES,), jnp.float32)
        for k in range(_K):
            acc = own_d[0] * gdot[slot, 1 + k, pl.ds(0, _LANES)]
            for ch in range(1, nchunk):
                acc = acc + own_d[ch] * gdot[
                    slot, 1 + k, pl.ds(ch * _LANES, _LANES)]
            dots = jnp.where(iot == k, jnp.sum(acc), dots)
        c0row = c0v[r, :]
        c1row = c1v[r, :]
        c2row = c2v[r, :]
        a = jnp.maximum(c0row + c1row * (dots - c2row), 0.0)
        oacc = [jnp.zeros((_LANES,), jnp.float32) for _ in range(nchunk)]
        for k in range(_K):
            ab = jnp.sum(jnp.where(iot == k, a, 0.0))
            for ch in range(nchunk):
                vec = gsum[slot, 1 + k, pl.ds(ch * _LANES, _LANES)]
                oacc[ch] = oacc[ch] + ab * vec
        @pl.when(r >= 2)
        def _():
            pltpu.make_async_copy(obuf.at[oslot], out_hbm.at[base + r - 2],
                                  sems[-2 + oslot]).wait()
        for ch in range(nchunk):
            own = gsum[slot, 0, pl.ds(ch * _LANES, _LANES)]
            res = _EPSILON * oacc[ch] + (1.0 - _EPSILON) * own
            obuf[oslot, pl.ds(ch * _LANES, _LANES)] = res
        pltpu.async_copy(obuf.at[oslot], out_hbm.at[base + r],
                         sems[-2 + oslot])

    for r in range(_NBUF - 1):
        fire(r, r)

    def step(i, carry):
        r0 = _NBUF * i
        for p in range(_NBUF):
            r = r0 + p

            @pl.when(r + _NBUF - 1 < rw)
            def _():
                fire(r + _NBUF - 1, (p - 1) % _NBUF)
            drain(r, p)
            compute_row(r, p, p % 2)
        return carry

    lax.fori_loop(0, rw // _NBUF, step, 0)
    for oslot in range(2):
        pltpu.make_async_copy(obuf.at[oslot],
                              out_hbm.at[base + rw - 2 + oslot],
                              sems[-2 + oslot]).wait()


def _sc_apply(sum_tab, dot_tab, idx, c0, c1, c2, shared_tables):
    bp, c = sum_tab.shape
    rw = bp // _NW
    mesh = plsc.VectorSubcoreMesh(core_axis_name="c", subcore_axis_name="s")
    scratch = [
        pltpu.VMEM((rw, _KW), jnp.int32),
        pltpu.VMEM((rw, _K), jnp.float32),
        pltpu.VMEM((rw, _K), jnp.float32),
        pltpu.VMEM((rw, _K), jnp.float32),
        pltpu.VMEM((2, c), jnp.float32),
        pltpu.VMEM((_NBUF, _KW, c), jnp.float32),
    ]
    if not shared_tables:
        scratch.append(pltpu.VMEM((_NBUF, _KW, c), jnp.float32))
    nsem = (_NBUF + 2) if shared_tables else (2 * _NBUF + 2)
    for _ in range(nsem):
        scratch.append(pltpu.SemaphoreType.DMA)

    def body(*refs):
        if shared_tables:
            (sum_hbm, idx_hbm, c0_hbm, c1_hbm, c2_hbm, out_hbm,
             idxv, c0v, c1v, c2v, obuf, gsum, *sems) = refs
            _sc_apply_body(True, c, rw,
                           (sum_hbm, idx_hbm, c0_hbm, c1_hbm, c2_hbm,
                            out_hbm, idxv, c0v, c1v, c2v, obuf, gsum,
                            list(sems)))
        else:
            (sum_hbm, dot_hbm, idx_hbm, c0_hbm, c1_hbm, c2_hbm, out_hbm,
             idxv, c0v, c1v, c2v, obuf, gsum, gdot, *sems) = refs
            _sc_apply_body(False, c, rw,
                           (sum_hbm, dot_hbm, idx_hbm, c0_hbm, c1_hbm,
                            c2_hbm, out_hbm, idxv, c0v, c1v, c2v, obuf,
                            gsum, gdot, list(sems)))

    kern = functools.partial(
        pl.kernel, mesh=mesh,
        out_type=jax.ShapeDtypeStruct((bp, c), jnp.float32),
        scratch_types=scratch,
        compiler_params=pltpu.CompilerParams(
            needs_layout_passes=False, use_tc_tiling_on_sc=False),
    )(body)
    if shared_tables:
        return kern(sum_tab, idx, c0, c1, c2)
    return kern(sum_tab, dot_tab, idx, c0, c1, c2)


def _pad_rows(a, bp):
    pad = [(0, bp - a.shape[0])] + [(0, 0)] * (a.ndim - 1)
    return jnp.pad(a, pad)


def kernel(x, L):
    h = x[0]
    b, c = h.shape
    bp = ((b + 8 * _NW - 1) // (8 * _NW)) * (8 * _NW)
    vals, idx, mnmx0 = _stage1(h, L)
    dval = vals[:, 1:_K + 1]
    dk = vals[:, _K + 1]
    gamma = jnp.mean(0.5 * (_K * dk - jnp.sum(dval, axis=1)))
    inv2g = 1.0 / (2.0 * gamma + 1e-8)
    eta = (1.0 / _K) * (1.0 + jnp.sum(dval, axis=1) * inv2g)

    c0 = eta[:, None] - inv2g * dval
    hp = _pad_rows(h, bp)
    own = jnp.arange(b, dtype=jnp.int32)[:, None]
    idxw = jnp.concatenate(
        [own, idx, jnp.broadcast_to(own, (b, _KW - _K - 1))], axis=1)
    idxp = _pad_rows(idxw, bp)
    c0p = _pad_rows(c0, bp)

    def consts(mnmx):
        c1 = inv2g * _LAM / (mnmx[:, 1] - mnmx[:, 0] + 1e-8)
        c1k = jnp.broadcast_to(c1[:, None], (b, _K))
        c2k = jnp.broadcast_to(mnmx[:, 0:1], (b, _K))
        return _pad_rows(c1k, bp), _pad_rows(c2k, bp)

    c1p, c2p = consts(mnmx0)
    fm1p = _sc_apply(hp, hp, idxp, c0p, c1p, c2p, shared_tables=True)

    mnmx1 = _stats(fm1p[:b])
    c1p, c2p = consts(mnmx1)
    fm2p = _sc_apply(hp, fm1p, idxp, c0p, c1p, c2p, shared_tables=False)
    return fm2p[:b][None, :, :]
